# baseline (speedup 1.0000x reference)
import functools

import jax
import jax.numpy as jnp
from jax import lax
from jax.experimental import pallas as pl
from jax.experimental.pallas import tpu as pltpu
from jax.experimental.pallas import tpu_sc as plsc

_SPARSITY = 0.5
_MOMENTUM = 0.1

_TBITS = 13
_MBITS = _TBITS - 8
_NBUCK = 1 << _TBITS
_SHF = 23 - _MBITS
_NEXP = 256
_NFINE = 1 << _MBITS
_UNROLL = 8


def _sc_kth_bits(x2, *, n_rows, length, k, num_ch):
    info = plsc.get_sparse_core_info()
    nc, ns = info.num_cores, info.num_subcores
    nw = nc * ns
    ch_per_w = num_ch // nw
    nl = n_rows * length
    mesh = plsc.VectorSubcoreMesh(core_axis_name="c", subcore_axis_name="s")

    @functools.partial(
        pl.kernel,
        mesh=mesh,
        out_type=jax.ShapeDtypeStruct((num_ch,), jnp.int32),
        scratch_types=[
            pltpu.VMEM((2 * nl,), jnp.float32),
            pltpu.VMEM((_NBUCK,), jnp.int32),
            pltpu.VMEM((ch_per_w,), jnp.int32),
            pltpu.SemaphoreType.DMA,
            pltpu.SemaphoreType.DMA,
        ],
        compiler_params=pltpu.CompilerParams(needs_layout_passes=False),
    )
    def body(x_hbm, kth_hbm, buf, hist, kbuf, sem_a, sem_b):
        wid = lax.axis_index("s") * nc + lax.axis_index("c")
        ch0 = wid * ch_per_w
        ones = jnp.ones((16,), jnp.int32)
        zeros16 = jnp.zeros((16,), jnp.int32)
        iota16 = lax.iota(jnp.int32, 16)
        lane0 = iota16 == 0
        stride32 = iota16 * _NFINE

        def z0(j, _):
            hist[pl.ds(j * 16, 16)] = zeros16
            return 0

        lax.fori_loop(0, _NBUCK // 16, z0, 0, unroll=_UNROLL)

        def dma_descs(c, slot, sem):
            return [
                pltpu.make_async_copy(
                    x_hbm.at[n * num_ch + c],
                    buf.at[pl.ds(slot * nl + n * length, length)],
                    sem,
                )
                for n in range(n_rows)
            ]

        def start_dma(c, slot, sem):
            for d in dma_descs(c, slot, sem):
                d.start()

        def wait_dma(c, slot, sem):
            for d in dma_descs(c, slot, sem):
                d.wait()

        def process(slot, i_local):
            base = slot * nl

            def p1(j, _):
                v = buf[pl.ds(base + j * 16, 16)]
                b = lax.shift_right_logical(
                    plsc.bitcast(v, jnp.int32), _SHF
                ) & jnp.int32(_NBUCK - 1)
                b = b ^ (lax.shift_right_logical(b, 5) & jnp.int32(31))
                plsc.addupdate_scatter(hist, [b], ones)
                return 0

            lax.fori_loop(0, nl // 16, p1, 0, unroll=_UNROLL)

            esel = jnp.int32(0)
            rank1 = jnp.int32(0)
            run = jnp.int32(0)
            for g in range(_NEXP // 16):
                acc = zeros16
                for f in range(_NFINE):
                    acc = acc + plsc.load_gather(
                        hist, [stride32 + jnp.int32(g * 16 * _NFINE + f)]
                    )
                cs = plsc.cumsum(acc)
                ind = (run + cs) < k
                esel = esel + jnp.sum(ind.astype(jnp.int32))
                rank1 = rank1 + jnp.sum(jnp.where(ind, acc, 0))
                run = run + jnp.sum(acc)

            kk = k - rank1
            fbase = esel * _NFINE
            exl = esel & jnp.int32(31)
            f0 = plsc.load_gather(hist, [fbase + (iota16 ^ exl)])
            f1 = plsc.load_gather(
                hist, [fbase + ((iota16 + jnp.int32(16)) ^ exl)]
            )
            cs0 = plsc.cumsum(f0)
            ind0 = cs0 < kk
            fsel = jnp.sum(ind0.astype(jnp.int32))
            run0 = jnp.sum(f0)
            cs1 = plsc.cumsum(f1)
            ind1 = (run0 + cs1) < kk
            fsel = fsel + jnp.sum(ind1.astype(jnp.int32))

            bits = (
                lax.shift_left(esel, 23)
                | lax.shift_left(fsel, _SHF)
                | jnp.int32(1 << (_SHF - 1))
            )
            plsc.store_scatter(
                kbuf, [jnp.full((16,), i_local, jnp.int32)],
                jnp.full((16,), bits, jnp.int32), mask=lane0,
            )

            def zz(j, _):
                hist[pl.ds(j * 16, 16)] = zeros16
                return 0

            lax.fori_loop(0, _NBUCK // 16, zz, 0, unroll=_UNROLL)

        start_dma(ch0, 0, sem_a)
        start_dma(ch0 + 1, 1, sem_b)

        def pair_body(i, _):
            ca = ch0 + 2 * i
            wait_dma(ca, 0, sem_a)
            process(0, 2 * i)

            @pl.when(i < ch_per_w // 2 - 1)
            def _prefetch_a():
                start_dma(ca + 2, 0, sem_a)

            wait_dma(ca + 1, 1, sem_b)
            process(1, 2 * i + 1)

            @pl.when(i < ch_per_w // 2 - 1)
            def _prefetch_b():
                start_dma(ca + 3, 1, sem_b)

            return 0

        lax.fori_loop(0, ch_per_w // 2, pair_body, 0)
        pltpu.sync_copy(kbuf, kth_hbm.at[pl.ds(ch0, ch_per_w)])

    return body(x2)


def _mask_body(x_ref, t_ref, kb_ref, o_ref):
    kth = jax.lax.bitcast_convert_type(kb_ref[...], jnp.float32)
    thr = t_ref[...] * (1.0 - _MOMENTUM) + kth * _MOMENTUM
    xv = x_ref[...]
    o_ref[...] = jnp.where(jnp.abs(xv) > thr[None, :, None], xv, 0.0)


def kernel(x, thresholds):
    N, C, L = x.shape
    k = max(1, int(N * L * _SPARSITY))
    kth_bits = _sc_kth_bits(
        x.reshape(N * C, L), n_rows=N, length=L, k=k, num_ch=C
    )
    c_chunk = 128
    out = pl.pallas_call(
        _mask_body,
        grid=(C // c_chunk,),
        in_specs=[
            pl.BlockSpec((N, c_chunk, L), lambda i: (0, i, 0)),
            pl.BlockSpec((c_chunk,), lambda i: (i,)),
            pl.BlockSpec((c_chunk,), lambda i: (i,)),
        ],
        out_specs=pl.BlockSpec((N, c_chunk, L), lambda i: (0, i, 0)),
        out_shape=jax.ShapeDtypeStruct((N, C, L), jnp.float32),
        compiler_params=pltpu.CompilerParams(
            dimension_semantics=("arbitrary",),
        ),
    )(x, thresholds, kth_bits)
    return out


# speedup vs baseline: 1.8550x; 1.8550x over previous
import functools

import jax
import jax.numpy as jnp
from jax import lax
from jax.experimental import pallas as pl
from jax.experimental.pallas import tpu as pltpu
from jax.experimental.pallas import tpu_sc as plsc

_SPARSITY = 0.5
_MOMENTUM = 0.1

_TBITS = 13
_MBITS = _TBITS - 8
_NBUCK = 1 << _TBITS
_SHF = 23 - _MBITS
_NEXP = 256
_NFINE = 1 << _MBITS
_UNROLL = 8


def _sc_kth_bits(x2, *, n_rows, length, k, num_ch, num_ch_sc):
    info = plsc.get_sparse_core_info()
    nc, ns = info.num_cores, info.num_subcores
    nw = nc * ns
    ch_per_w = num_ch_sc // nw
    nl = n_rows * length
    mesh = plsc.VectorSubcoreMesh(core_axis_name="c", subcore_axis_name="s")

    @functools.partial(
        pl.kernel,
        mesh=mesh,
        out_type=jax.ShapeDtypeStruct((num_ch_sc,), jnp.int32),
        scratch_types=[
            pltpu.VMEM((2 * nl,), jnp.float32),
            pltpu.VMEM((_NBUCK,), jnp.int32),
            pltpu.VMEM((ch_per_w,), jnp.int32),
            pltpu.SemaphoreType.DMA,
            pltpu.SemaphoreType.DMA,
        ],
        compiler_params=pltpu.CompilerParams(needs_layout_passes=False),
    )
    def body(x_hbm, kth_hbm, buf, hist, kbuf, sem_a, sem_b):
        wid = lax.axis_index("s") * nc + lax.axis_index("c")
        ch0 = wid * ch_per_w
        ones = jnp.ones((16,), jnp.int32)
        zeros16 = jnp.zeros((16,), jnp.int32)
        iota16 = lax.iota(jnp.int32, 16)
        lane0 = iota16 == 0
        stride32 = iota16 * _NFINE

        def z0(j, _):
            hist[pl.ds(j * 16, 16)] = zeros16
            return 0

        lax.fori_loop(0, _NBUCK // 16, z0, 0, unroll=_UNROLL)

        def dma_descs(c, slot, sem):
            return [
                pltpu.make_async_copy(
                    x_hbm.at[n * num_ch + c],
                    buf.at[pl.ds(slot * nl + n * length, length)],
                    sem,
                )
                for n in range(n_rows)
            ]

        def start_dma(c, slot, sem):
            for d in dma_descs(c, slot, sem):
                d.start()

        def wait_dma(c, slot, sem):
            for d in dma_descs(c, slot, sem):
                d.wait()

        def process(slot, i_local):
            base = slot * nl

            def p1(j, _):
                v = buf[pl.ds(base + j * 16, 16)]
                b = lax.shift_right_logical(
                    plsc.bitcast(v, jnp.int32), _SHF
                ) & jnp.int32(_NBUCK - 1)
                plsc.addupdate_scatter(hist, [b], ones)
                return 0

            lax.fori_loop(0, nl // 16, p1, 0, unroll=_UNROLL)

            esel = jnp.int32(0)
            rank1 = jnp.int32(0)
            run = jnp.int32(0)
            for g in range(_NEXP // 16):
                acc = zeros16
                for f in range(_NFINE):
                    acc = acc + plsc.load_gather(
                        hist, [stride32 + jnp.int32(g * 16 * _NFINE + f)]
                    )
                cs = plsc.cumsum(acc)
                ind = (run + cs) < k
                esel = esel + jnp.sum(ind.astype(jnp.int32))
                rank1 = rank1 + jnp.sum(jnp.where(ind, acc, 0))
                run = run + jnp.sum(acc)

            kk = k - rank1
            fbase = esel * _NFINE
            f0 = plsc.load_gather(hist, [fbase + iota16])
            f1 = plsc.load_gather(hist, [fbase + jnp.int32(16) + iota16])
            cs0 = plsc.cumsum(f0)
            ind0 = cs0 < kk
            fsel = jnp.sum(ind0.astype(jnp.int32))
            run0 = jnp.sum(f0)
            cs1 = plsc.cumsum(f1)
            ind1 = (run0 + cs1) < kk
            fsel = fsel + jnp.sum(ind1.astype(jnp.int32))

            bits = (
                lax.shift_left(esel, 23)
                | lax.shift_left(fsel, _SHF)
                | jnp.int32(1 << (_SHF - 1))
            )
            plsc.store_scatter(
                kbuf, [jnp.full((16,), i_local, jnp.int32)],
                jnp.full((16,), bits, jnp.int32), mask=lane0,
            )

            def zz(j, _):
                hist[pl.ds(j * 16, 16)] = zeros16
                return 0

            lax.fori_loop(0, _NBUCK // 16, zz, 0, unroll=_UNROLL)

        start_dma(ch0, 0, sem_a)
        start_dma(ch0 + 1, 1, sem_b)

        def pair_body(i, _):
            ca = ch0 + 2 * i
            wait_dma(ca, 0, sem_a)
            process(0, 2 * i)

            @pl.when(i < ch_per_w // 2 - 1)
            def _prefetch_a():
                start_dma(ca + 2, 0, sem_a)

            wait_dma(ca + 1, 1, sem_b)
            process(1, 2 * i + 1)

            @pl.when(i < ch_per_w // 2 - 1)
            def _prefetch_b():
                start_dma(ca + 3, 1, sem_b)

            return 0

        lax.fori_loop(0, ch_per_w // 2, pair_body, 0)
        pltpu.sync_copy(kbuf, kth_hbm.at[pl.ds(ch0, ch_per_w)])

    return body(x2)


_NIT = 16


def _tc_rank_body(x_ref, o_ref, bits_ref, p_ref, *, k):
    j = pl.program_id(1)

    @pl.when(j == 0)
    def _init():
        bits_ref[...] = (
            jax.lax.bitcast_convert_type(x_ref[...], jnp.int32)
            & jnp.int32(0x7FFFFFFF)
        )
        p_ref[...] = jnp.zeros_like(p_ref)

    @pl.when(j < _NIT)
    def _search():
        bit = jnp.int32(1) << (30 - j)
        cand = p_ref[...] | bit
        cmp = (bits_ref[...] < cand[None, :, None]).astype(jnp.int32)
        cnt = jnp.sum(cmp, axis=(0, 2))
        p_ref[...] = jnp.where(cnt >= k, p_ref[...], cand)

    @pl.when(j == _NIT)
    def _finalize():
        o_ref[...] = p_ref[...] + (jnp.int32(1) << (30 - _NIT))


def _tc_kth_bits(x, *, k, ch_lo, c_chunk):
    N, C, L = x.shape
    n_chunks = (C - ch_lo) // c_chunk
    off = ch_lo // c_chunk
    return pl.pallas_call(
        functools.partial(_tc_rank_body, k=k),
        grid=(n_chunks, _NIT + 1),
        in_specs=[
            pl.BlockSpec((N, c_chunk, L), lambda i, j: (0, i + off, 0)),
        ],
        out_specs=pl.BlockSpec((c_chunk,), lambda i, j: (i,)),
        out_shape=jax.ShapeDtypeStruct((C - ch_lo,), jnp.int32),
        scratch_shapes=[
            pltpu.VMEM((N, c_chunk, L), jnp.int32),
            pltpu.VMEM((c_chunk,), jnp.int32),
        ],
        compiler_params=pltpu.CompilerParams(
            dimension_semantics=("arbitrary", "arbitrary"),
        ),
    )(x)


def _mask_body(x_ref, t_ref, kb_ref, o_ref):
    kth = jax.lax.bitcast_convert_type(kb_ref[...], jnp.float32)
    thr = t_ref[...] * (1.0 - _MOMENTUM) + kth * _MOMENTUM
    xv = x_ref[...]
    o_ref[...] = jnp.where(jnp.abs(xv) > thr[None, :, None], xv, 0.0)


def kernel(x, thresholds):
    N, C, L = x.shape
    k = max(1, int(N * L * _SPARSITY))
    c_sc = C // 2
    sc_bits = _sc_kth_bits(
        x.reshape(N * C, L), n_rows=N, length=L, k=k, num_ch=C,
        num_ch_sc=c_sc,
    )
    tc_bits = _tc_kth_bits(x, k=k, ch_lo=c_sc, c_chunk=128)
    kth_bits = jnp.concatenate([sc_bits, tc_bits])
    c_chunk = 128
    out = pl.pallas_call(
        _mask_body,
        grid=(C // c_chunk,),
        in_specs=[
            pl.BlockSpec((N, c_chunk, L), lambda i: (0, i, 0)),
            pl.BlockSpec((c_chunk,), lambda i: (i,)),
            pl.BlockSpec((c_chunk,), lambda i: (i,)),
        ],
        out_specs=pl.BlockSpec((N, c_chunk, L), lambda i: (0, i, 0)),
        out_shape=jax.ShapeDtypeStruct((N, C, L), jnp.float32),
        compiler_params=pltpu.CompilerParams(
            dimension_semantics=("arbitrary",),
        ),
    )(x, thresholds, kth_bits)
    return out


# speedup vs baseline: 2.0236x; 1.0909x over previous
import functools

import jax
import jax.numpy as jnp
from jax import lax
from jax.experimental import pallas as pl
from jax.experimental.pallas import tpu as pltpu
from jax.experimental.pallas import tpu_sc as plsc

_SPARSITY = 0.5
_MOMENTUM = 0.1

_TBITS = 13
_MBITS = _TBITS - 8
_NBUCK = 1 << _TBITS
_SHF = 23 - _MBITS
_NEXP = 256
_NFINE = 1 << _MBITS
_UNROLL = 8


def _sc_kth_bits(x2, *, n_rows, length, k, num_ch, num_ch_sc):
    info = plsc.get_sparse_core_info()
    nc, ns = info.num_cores, info.num_subcores
    nw = nc * ns
    ch_per_w = num_ch_sc // nw
    nl = n_rows * length
    mesh = plsc.VectorSubcoreMesh(core_axis_name="c", subcore_axis_name="s")

    @functools.partial(
        pl.kernel,
        mesh=mesh,
        out_type=jax.ShapeDtypeStruct((num_ch_sc,), jnp.int32),
        scratch_types=[
            pltpu.VMEM((2 * nl,), jnp.float32),
            pltpu.VMEM((_NBUCK,), jnp.int32),
            pltpu.VMEM((ch_per_w,), jnp.int32),
            pltpu.SemaphoreType.DMA,
            pltpu.SemaphoreType.DMA,
        ],
        compiler_params=pltpu.CompilerParams(needs_layout_passes=False),
    )
    def body(x_hbm, kth_hbm, buf, hist, kbuf, sem_a, sem_b):
        wid = lax.axis_index("s") * nc + lax.axis_index("c")
        ch0 = wid * ch_per_w
        ones = jnp.ones((16,), jnp.int32)
        zeros16 = jnp.zeros((16,), jnp.int32)
        iota16 = lax.iota(jnp.int32, 16)
        lane0 = iota16 == 0
        stride32 = iota16 * _NFINE

        def z0(j, _):
            hist[pl.ds(j * 16, 16)] = zeros16
            return 0

        lax.fori_loop(0, _NBUCK // 16, z0, 0, unroll=_UNROLL)

        def dma_descs(c, slot, sem):
            return [
                pltpu.make_async_copy(
                    x_hbm.at[n * num_ch + c],
                    buf.at[pl.ds(slot * nl + n * length, length)],
                    sem,
                )
                for n in range(n_rows)
            ]

        def start_dma(c, slot, sem):
            for d in dma_descs(c, slot, sem):
                d.start()

        def wait_dma(c, slot, sem):
            for d in dma_descs(c, slot, sem):
                d.wait()

        def process(slot, i_local):
            base = slot * nl

            def p1(j, _):
                v = buf[pl.ds(base + j * 16, 16)]
                b = lax.shift_right_logical(
                    plsc.bitcast(v, jnp.int32), _SHF
                ) & jnp.int32(_NBUCK - 1)
                plsc.addupdate_scatter(hist, [b], ones)
                return 0

            lax.fori_loop(0, nl // 16, p1, 0, unroll=_UNROLL)

            esel = jnp.int32(0)
            rank1 = jnp.int32(0)
            run = jnp.int32(0)
            for g in range(_NEXP // 16):
                acc = zeros16
                for f in range(_NFINE):
                    acc = acc + plsc.load_gather(
                        hist, [stride32 + jnp.int32(g * 16 * _NFINE + f)]
                    )
                cs = plsc.cumsum(acc)
                ind = (run + cs) < k
                esel = esel + jnp.sum(ind.astype(jnp.int32))
                rank1 = rank1 + jnp.sum(jnp.where(ind, acc, 0))
                run = run + jnp.sum(acc)

            kk = k - rank1
            fbase = esel * _NFINE
            f0 = plsc.load_gather(hist, [fbase + iota16])
            f1 = plsc.load_gather(hist, [fbase + jnp.int32(16) + iota16])
            cs0 = plsc.cumsum(f0)
            ind0 = cs0 < kk
            fsel = jnp.sum(ind0.astype(jnp.int32))
            run0 = jnp.sum(f0)
            cs1 = plsc.cumsum(f1)
            ind1 = (run0 + cs1) < kk
            fsel = fsel + jnp.sum(ind1.astype(jnp.int32))

            bits = (
                lax.shift_left(esel, 23)
                | lax.shift_left(fsel, _SHF)
                | jnp.int32(1 << (_SHF - 1))
            )
            plsc.store_scatter(
                kbuf, [jnp.full((16,), i_local, jnp.int32)],
                jnp.full((16,), bits, jnp.int32), mask=lane0,
            )

            def zz(j, _):
                hist[pl.ds(j * 16, 16)] = zeros16
                return 0

            lax.fori_loop(0, _NBUCK // 16, zz, 0, unroll=_UNROLL)

        start_dma(ch0, 0, sem_a)
        start_dma(ch0 + 1, 1, sem_b)

        def pair_body(i, _):
            ca = ch0 + 2 * i
            wait_dma(ca, 0, sem_a)
            process(0, 2 * i)

            @pl.when(i < ch_per_w // 2 - 1)
            def _prefetch_a():
                start_dma(ca + 2, 0, sem_a)

            wait_dma(ca + 1, 1, sem_b)
            process(1, 2 * i + 1)

            @pl.when(i < ch_per_w // 2 - 1)
            def _prefetch_b():
                start_dma(ca + 3, 1, sem_b)

            return 0

        lax.fori_loop(0, ch_per_w // 2, pair_body, 0)
        pltpu.sync_copy(kbuf, kth_hbm.at[pl.ds(ch0, ch_per_w)])

    return body(x2)


_NIT = 16


def _tc_rank_body(x_ref, o_ref, bits_ref, p_ref, *, k):
    j = pl.program_id(1)

    @pl.when(j == 0)
    def _init():
        bits_ref[...] = (
            jax.lax.bitcast_convert_type(x_ref[...], jnp.int32)
            & jnp.int32(0x7FFFFFFF)
        )
        p_ref[...] = jnp.zeros_like(p_ref)

    @pl.when(j < _NIT)
    def _search():
        bit = jnp.int32(1) << (30 - j)
        cand = p_ref[...] | bit
        cmp = (bits_ref[...] < cand[None, :, None]).astype(jnp.int32)
        cnt = jnp.sum(cmp, axis=(0, 2))
        p_ref[...] = jnp.where(cnt >= k, p_ref[...], cand)

    @pl.when(j == _NIT)
    def _finalize():
        o_ref[...] = p_ref[...] + (jnp.int32(1) << (30 - _NIT))


def _tc_kth_bits(x, *, k, ch_lo, c_chunk):
    N, C, L = x.shape
    n_chunks = (C - ch_lo) // c_chunk
    off = ch_lo // c_chunk
    return pl.pallas_call(
        functools.partial(_tc_rank_body, k=k),
        grid=(n_chunks, _NIT + 1),
        in_specs=[
            pl.BlockSpec((N, c_chunk, L), lambda i, j: (0, i + off, 0)),
        ],
        out_specs=pl.BlockSpec((c_chunk,), lambda i, j: (i,)),
        out_shape=jax.ShapeDtypeStruct((C - ch_lo,), jnp.int32),
        scratch_shapes=[
            pltpu.VMEM((N, c_chunk, L), jnp.int32),
            pltpu.VMEM((c_chunk,), jnp.int32),
        ],
        compiler_params=pltpu.CompilerParams(
            dimension_semantics=("arbitrary", "arbitrary"),
        ),
    )(x)


def _mask_body(x_ref, t_ref, kb_ref, o_ref):
    kth = jax.lax.bitcast_convert_type(kb_ref[...], jnp.float32)
    thr = t_ref[...] * (1.0 - _MOMENTUM) + kth * _MOMENTUM
    xv = x_ref[...]
    o_ref[...] = jnp.where(jnp.abs(xv) > thr[None, :, None], xv, 0.0)


def _mask_body_aliased(prev_ref, x_ref, t_ref, kb_ref, o_ref):
    del prev_ref
    _mask_body(x_ref, t_ref, kb_ref, o_ref)


def _mask_range(x, thresholds, bits, ch_lo, ch_hi, prev=None):
    N, C, L = x.shape
    cc = 128
    off = ch_lo // cc
    grid = ((ch_hi - ch_lo) // cc,)
    in_specs = [
        pl.BlockSpec((N, cc, L), lambda i: (0, i + off, 0)),
        pl.BlockSpec((cc,), lambda i: (i + off,)),
        pl.BlockSpec((cc,), lambda i: (i,)),
    ]
    args = (x, thresholds, bits)
    body = _mask_body
    kwargs = {}
    if prev is not None:
        in_specs = [
            pl.BlockSpec(memory_space=pltpu.MemorySpace.HBM)
        ] + in_specs
        args = (prev,) + args
        body = _mask_body_aliased
        kwargs = dict(input_output_aliases={0: 0})
    return pl.pallas_call(
        body,
        grid=grid,
        in_specs=in_specs,
        out_specs=pl.BlockSpec((N, cc, L), lambda i: (0, i + off, 0)),
        out_shape=jax.ShapeDtypeStruct((N, C, L), jnp.float32),
        compiler_params=pltpu.CompilerParams(
            dimension_semantics=("arbitrary",),
        ),
        **kwargs,
    )(*args)


def kernel(x, thresholds):
    N, C, L = x.shape
    k = max(1, int(N * L * _SPARSITY))
    c_sc = C // 2
    sc_bits = _sc_kth_bits(
        x.reshape(N * C, L), n_rows=N, length=L, k=k, num_ch=C,
        num_ch_sc=c_sc,
    )
    tc_bits = _tc_kth_bits(x, k=k, ch_lo=c_sc, c_chunk=128)
    out_tc = _mask_range(x, thresholds, tc_bits, c_sc, C)
    out = _mask_range(x, thresholds, sc_bits, 0, c_sc, prev=out_tc)
    return out
